# baseline (speedup 1.0000x reference)
import jax
import jax.numpy as jnp
from jax import lax
from jax.experimental import pallas as pl
from jax.experimental.pallas import tpu as pltpu
from jax.experimental.pallas import tpu_sc as plsc

NC = 2
NS = 16
LANE = 64
GRP = 16
NBUF = 4


def _sc_agg(n_pad: int, width: int, nsteps: int):
  rows_per_tile = n_pad // NS
  ngroups = nsteps // GRP
  mesh = plsc.VectorSubcoreMesh(
      core_axis_name="c", subcore_axis_name="s", num_cores=NC, num_subcores=NS)

  def body(x_hbm, src_hbm, dst_hbm, out_hbm, src_r, dst_r, *rest):
    bufs = rest[:NBUF]
    sem_g = rest[NBUF:2 * NBUF]
    sem_s = rest[2 * NBUF:3 * NBUF]
    acc = rest[3 * NBUF]
    c = lax.axis_index("c")
    s = lax.axis_index("s")
    wid = c * NS + s
    base = s * rows_per_tile

    zv = jnp.zeros((16,), jnp.float32)

    @pl.loop(0, LANE)
    def _zero_buf(i):
      for j in range(width // 16):
        bufs[0][i, pl.ds(j * 16, 16)] = zv

    @pl.loop(0, rows_per_tile // LANE)
    def _zero_acc(q):
      pltpu.sync_copy(bufs[0], acc.at[pl.ds(base + q * LANE, LANE)])

    pltpu.sync_copy(src_hbm.at[wid, pl.ds(0, GRP)], src_r.at[0])
    pltpu.sync_copy(dst_hbm.at[wid, pl.ds(0, GRP)], dst_r.at[0])
    pltpu.sync_copy(src_hbm.at[wid, pl.ds(GRP, GRP)], src_r.at[1])
    pltpu.sync_copy(dst_hbm.at[wid, pl.ds(GRP, GRP)], dst_r.at[1])

    for t in range(NBUF - 1):
      pltpu.async_copy(x_hbm.at[src_r.at[0, t]], bufs[t], sem_g[t])
    plsc.subcore_barrier()

    @pl.loop(0, ngroups)
    def _group(g):
      b = g % 2
      nb = 1 - b
      for t in range(GRP):
        buf = bufs[t % NBUF]
        pltpu.make_async_copy(
            x_hbm.at[src_r.at[b, t]], buf, sem_g[t % NBUF]).wait()
        pltpu.async_copy(buf, acc.at[dst_r.at[b, t]], sem_s[t % NBUF],
                         add=True)
        if t > 0:
          pbuf = bufs[(t - 1) % NBUF]
          pltpu.make_async_copy(
              pbuf, acc.at[dst_r.at[b, t - 1]], sem_s[(t - 1) % NBUF]).wait()
          if t + NBUF - 1 < GRP:
            pltpu.async_copy(
                x_hbm.at[src_r.at[b, t + NBUF - 1]], pbuf,
                sem_g[(t - 1) % NBUF])
          else:
            @pl.when(g < ngroups - 1)
            def _start_next():
              pltpu.async_copy(
                  x_hbm.at[src_r.at[nb, t + NBUF - 1 - GRP]], pbuf,
                  sem_g[(t - 1) % NBUF])
        else:
          pltpu.async_copy(
              x_hbm.at[src_r.at[b, NBUF - 1]], bufs[NBUF - 1],
              sem_g[NBUF - 1])

      pltpu.make_async_copy(
          bufs[(GRP - 1) % NBUF], acc.at[dst_r.at[b, GRP - 1]],
          sem_s[(GRP - 1) % NBUF]).wait()

      @pl.when(g + 2 < ngroups)
      def _refill():
        off = (g + 2) * GRP
        pltpu.sync_copy(src_hbm.at[wid, pl.ds(off, GRP)], src_r.at[b])
        pltpu.sync_copy(dst_hbm.at[wid, pl.ds(off, GRP)], dst_r.at[b])

    plsc.subcore_barrier()

    @pl.loop(0, rows_per_tile // LANE)
    def _publish(q):
      pltpu.sync_copy(acc.at[pl.ds(base + q * LANE, LANE)], bufs[0])
      pltpu.sync_copy(bufs[0], out_hbm.at[c, pl.ds(base + q * LANE, LANE)])

  return pl.kernel(
      body,
      out_type=jax.ShapeDtypeStruct((NC, n_pad, width), jnp.float32),
      mesh=mesh,
      scratch_types=(
          [
              pltpu.VMEM((2, GRP, LANE), jnp.int32),
              pltpu.VMEM((2, GRP, LANE), jnp.int32),
          ]
          + [pltpu.VMEM((LANE, width), jnp.float32) for _ in range(NBUF)]
          + [pltpu.SemaphoreType.DMA] * (2 * NBUF)
          + [pltpu.VMEM_SHARED((n_pad, width), jnp.float32)]
      ),
  )


def _sc_cnt(n_pad: int, width: int, nsteps: int):
  rows_per_tile = n_pad // NS
  mesh = plsc.VectorSubcoreMesh(
      core_axis_name="c", subcore_axis_name="s", num_cores=NC, num_subcores=NS)

  def body(dst_hbm, out_hbm, dst_v, zbuf, obuf, acc, sem):
    c = lax.axis_index("c")
    s = lax.axis_index("s")
    wid = c * NS + s
    base = s * rows_per_tile

    zv = jnp.zeros((16,), jnp.float32)
    ov = jnp.ones((16,), jnp.float32)

    @pl.loop(0, LANE)
    def _fill(i):
      for j in range(width // 16):
        zbuf[i, pl.ds(j * 16, 16)] = zv
        obuf[i, pl.ds(j * 16, 16)] = ov

    @pl.loop(0, rows_per_tile // LANE)
    def _zero_acc(q):
      pltpu.sync_copy(zbuf, acc.at[pl.ds(base + q * LANE, LANE)])

    pltpu.sync_copy(dst_hbm.at[wid], dst_v)
    plsc.subcore_barrier()

    @pl.loop(0, nsteps)
    def _scatter(j):
      pltpu.async_copy(obuf, acc.at[dst_v.at[j]], sem, add=True)

    @pl.loop(0, nsteps)
    def _drain(j):
      pltpu.make_async_copy(obuf, acc.at[dst_v.at[j]], sem).wait()

    plsc.subcore_barrier()

    @pl.loop(0, rows_per_tile // LANE)
    def _publish(q):
      pltpu.sync_copy(acc.at[pl.ds(base + q * LANE, LANE)], zbuf)
      pltpu.sync_copy(zbuf, out_hbm.at[c, pl.ds(base + q * LANE, LANE)])

  return pl.kernel(
      body,
      out_type=jax.ShapeDtypeStruct((NC, n_pad, width), jnp.float32),
      mesh=mesh,
      scratch_types=[
          pltpu.VMEM((nsteps, LANE), jnp.int32),
          pltpu.VMEM((LANE, width), jnp.float32),
          pltpu.VMEM((LANE, width), jnp.float32),
          pltpu.VMEM_SHARED((n_pad, width), jnp.float32),
          pltpu.SemaphoreType.DMA,
      ],
  )


def _dot_t(a, w):
  return lax.dot_general(a, w, (((1,), (1,)), ((), ())),
                         preferred_element_type=jnp.float32)


def _tc1_body(p_ref, c_ref, x_ref, wl_ref, wr_ref, b_ref, h_ref, inv_ref):
  p = p_ref[...]
  agg = p[0] + p[1]
  cnt = c_ref[0, :, 0:1] + c_ref[1, :, 0:1]
  inv = 1.0 / jnp.maximum(cnt, 1.0)
  inv_ref[...] = inv
  mean = agg * inv
  h = _dot_t(mean, wl_ref[...]) + b_ref[...] + _dot_t(x_ref[...], wr_ref[...])
  h_ref[...] = jnp.maximum(h, 0.0)


def _tc2_body(p_ref, inv_ref, h_ref, wl_ref, wr_ref, b_ref, o_ref):
  p = p_ref[...]
  agg = p[0] + p[1]
  mean = agg * inv_ref[...]
  z = _dot_t(mean, wl_ref[...]) + b_ref[...] + _dot_t(h_ref[...], wr_ref[...])
  z = jnp.maximum(z, 0.0)
  m = jnp.max(z, axis=1, keepdims=True)
  lse = jnp.log(jnp.sum(jnp.exp(z - m), axis=1, keepdims=True)) + m
  o_ref[...] = z - lse


def kernel(x, edge_index, W1_l, b1_l, W1_r, W2_l, b2_l, W2_r):
  n, d = x.shape
  e = edge_index.shape[1]
  nw = NC * NS

  nsteps = -(-e // (nw * LANE * GRP)) * GRP
  e_pad = nw * nsteps * LANE
  n_pad = -(-(n + 1) // (NS * LANE)) * (NS * LANE)

  src = edge_index[0].astype(jnp.int32)
  dst = edge_index[1].astype(jnp.int32)
  pad_idx = jnp.arange(e_pad - e, dtype=jnp.int32)
  src = jnp.concatenate([src, pad_idx % n])
  dst = jnp.concatenate([dst, n + pad_idx % (n_pad - n)])
  src3 = src.reshape(nw, nsteps, LANE)
  dst3 = dst.reshape(nw, nsteps, LANE)

  parts1 = _sc_agg(n_pad, d, nsteps)(x, src3, dst3)
  cnt_parts = _sc_cnt(n_pad, d, nsteps)(dst3)

  rb = 2048
  grid = (-(-n // rb),)
  b1_2d = b1_l.reshape(1, d)
  b2_2d = b2_l.reshape(1, d)

  h, invcol = pl.pallas_call(
      _tc1_body,
      grid=grid,
      in_specs=[
          pl.BlockSpec((NC, rb, d), lambda j: (0, j, 0)),
          pl.BlockSpec((NC, rb, d), lambda j: (0, j, 0)),
          pl.BlockSpec((rb, d), lambda j: (j, 0)),
          pl.BlockSpec((d, d), lambda j: (0, 0)),
          pl.BlockSpec((d, d), lambda j: (0, 0)),
          pl.BlockSpec((1, d), lambda j: (0, 0)),
      ],
      out_specs=[
          pl.BlockSpec((rb, d), lambda j: (j, 0)),
          pl.BlockSpec((rb, 1), lambda j: (j, 0)),
      ],
      out_shape=[
          jax.ShapeDtypeStruct((n, d), jnp.float32),
          jax.ShapeDtypeStruct((n_pad, 1), jnp.float32),
      ],
  )(parts1, cnt_parts, x, W1_l, W1_r, b1_2d)

  parts2 = _sc_agg(n_pad, d, nsteps)(h, src3, dst3)

  out = pl.pallas_call(
      _tc2_body,
      grid=grid,
      in_specs=[
          pl.BlockSpec((NC, rb, d), lambda j: (0, j, 0)),
          pl.BlockSpec((rb, 1), lambda j: (j, 0)),
          pl.BlockSpec((rb, d), lambda j: (j, 0)),
          pl.BlockSpec((d, d), lambda j: (0, 0)),
          pl.BlockSpec((d, d), lambda j: (0, 0)),
          pl.BlockSpec((1, d), lambda j: (0, 0)),
      ],
      out_specs=pl.BlockSpec((rb, d), lambda j: (j, 0)),
      out_shape=jax.ShapeDtypeStruct((n, d), jnp.float32),
  )(parts2, invcol, h, W2_l, W2_r, b2_2d)

  return out


# speedup vs baseline: 1.0024x; 1.0024x over previous
import jax
import jax.numpy as jnp
from jax import lax
from jax.experimental import pallas as pl
from jax.experimental.pallas import tpu as pltpu
from jax.experimental.pallas import tpu_sc as plsc

NC = 2
NS = 16
LANE = 64
GRP = 16
NBUF = 4


def _sc_agg(n_pad: int, width: int, nsteps: int):
  rows_per_tile = n_pad // NS
  ngroups = nsteps // GRP
  mesh = plsc.VectorSubcoreMesh(
      core_axis_name="c", subcore_axis_name="s", num_cores=NC, num_subcores=NS)

  def body(x_hbm, src_hbm, dst_hbm, out_hbm, src_r, dst_r, *rest):
    bufs = rest[:NBUF]
    sem_g = rest[NBUF:2 * NBUF]
    sem_s = rest[2 * NBUF:3 * NBUF]
    acc = rest[3 * NBUF]
    c = lax.axis_index("c")
    s = lax.axis_index("s")
    wid = c * NS + s
    base = s * rows_per_tile

    zv = jnp.zeros((16,), jnp.float32)

    @pl.loop(0, LANE)
    def _zero_buf(i):
      for j in range(width // 16):
        bufs[0][i, pl.ds(j * 16, 16)] = zv

    @pl.loop(0, rows_per_tile // LANE)
    def _zero_acc(q):
      pltpu.sync_copy(bufs[0], acc.at[pl.ds(base + q * LANE, LANE)])

    pltpu.sync_copy(src_hbm.at[wid, pl.ds(0, GRP)], src_r.at[0])
    pltpu.sync_copy(dst_hbm.at[wid, pl.ds(0, GRP)], dst_r.at[0])
    pltpu.sync_copy(src_hbm.at[wid, pl.ds(GRP, GRP)], src_r.at[1])
    pltpu.sync_copy(dst_hbm.at[wid, pl.ds(GRP, GRP)], dst_r.at[1])

    for t in range(NBUF - 1):
      pltpu.async_copy(x_hbm.at[src_r.at[0, t]], bufs[t], sem_g[t])
    plsc.subcore_barrier()

    @pl.loop(0, ngroups)
    def _group(g):
      b = g % 2
      nb = 1 - b
      for t in range(GRP):
        buf = bufs[t % NBUF]
        pltpu.make_async_copy(
            x_hbm.at[src_r.at[b, t]], buf, sem_g[t % NBUF]).wait()
        pltpu.async_copy(buf, acc.at[dst_r.at[b, t]], sem_s[t % NBUF],
                         add=True)
        if t > 0:
          pbuf = bufs[(t - 1) % NBUF]
          pltpu.make_async_copy(
              pbuf, acc.at[dst_r.at[b, t - 1]], sem_s[(t - 1) % NBUF]).wait()
          if t + NBUF - 1 < GRP:
            pltpu.async_copy(
                x_hbm.at[src_r.at[b, t + NBUF - 1]], pbuf,
                sem_g[(t - 1) % NBUF])
          else:
            @pl.when(g < ngroups - 1)
            def _start_next():
              pltpu.async_copy(
                  x_hbm.at[src_r.at[nb, t + NBUF - 1 - GRP]], pbuf,
                  sem_g[(t - 1) % NBUF])
        else:
          pltpu.async_copy(
              x_hbm.at[src_r.at[b, NBUF - 1]], bufs[NBUF - 1],
              sem_g[NBUF - 1])

      pltpu.make_async_copy(
          bufs[(GRP - 1) % NBUF], acc.at[dst_r.at[b, GRP - 1]],
          sem_s[(GRP - 1) % NBUF]).wait()

      @pl.when(g + 2 < ngroups)
      def _refill():
        off = (g + 2) * GRP
        pltpu.sync_copy(src_hbm.at[wid, pl.ds(off, GRP)], src_r.at[b])
        pltpu.sync_copy(dst_hbm.at[wid, pl.ds(off, GRP)], dst_r.at[b])

    plsc.subcore_barrier()

    @pl.loop(0, rows_per_tile // LANE)
    def _publish(q):
      pltpu.sync_copy(acc.at[pl.ds(base + q * LANE, LANE)], bufs[0])
      pltpu.sync_copy(bufs[0], out_hbm.at[c, pl.ds(base + q * LANE, LANE)])

  return pl.kernel(
      body,
      out_type=jax.ShapeDtypeStruct((NC, n_pad, width), jnp.float32),
      mesh=mesh,
      scratch_types=(
          [
              pltpu.VMEM((2, GRP, LANE), jnp.int32),
              pltpu.VMEM((2, GRP, LANE), jnp.int32),
          ]
          + [pltpu.VMEM((LANE, width), jnp.float32) for _ in range(NBUF)]
          + [pltpu.SemaphoreType.DMA] * (2 * NBUF)
          + [pltpu.VMEM_SHARED((n_pad, width), jnp.float32)]
      ),
  )


def _sc_cnt_agg(n_pad: int, width: int, nsteps: int):
  rows_per_tile = n_pad // NS
  ngroups = nsteps // GRP
  mesh = plsc.VectorSubcoreMesh(
      core_axis_name="c", subcore_axis_name="s", num_cores=NC, num_subcores=NS)

  def body(x_hbm, src_hbm, dst_hbm, out_hbm, cnt_hbm, src_r, dst_r, *rest):
    bufs = rest[:NBUF]
    sem_g = rest[NBUF:2 * NBUF]
    sem_s = rest[2 * NBUF:3 * NBUF]
    acc = rest[3 * NBUF]
    c = lax.axis_index("c")
    s = lax.axis_index("s")
    wid = c * NS + s
    base = s * rows_per_tile

    zv = jnp.zeros((16,), jnp.float32)
    ov = jnp.ones((16,), jnp.float32)

    @pl.loop(0, LANE)
    def _fill(i):
      for j in range(width // 16):
        bufs[0][i, pl.ds(j * 16, 16)] = zv
        bufs[1][i, pl.ds(j * 16, 16)] = ov

    @pl.loop(0, rows_per_tile // LANE)
    def _zero_acc(q):
      pltpu.sync_copy(bufs[0], acc.at[pl.ds(base + q * LANE, LANE)])

    pltpu.sync_copy(dst_hbm.at[wid, pl.ds(0, GRP)], dst_r.at[0])
    pltpu.sync_copy(dst_hbm.at[wid, pl.ds(GRP, GRP)], dst_r.at[1])
    plsc.subcore_barrier()

    @pl.loop(0, ngroups // 2)
    def _cnt_pair(k):
      for half in (0, 1):
        @pl.when(k >= 1)
        def _recycle(half=half):
          for t in range(GRP):
            pltpu.make_async_copy(
                bufs[1], acc.at[dst_r.at[half, t]], sem_s[half]).wait()
          pltpu.sync_copy(
              dst_hbm.at[wid, pl.ds((2 * k + half) * GRP, GRP)],
              dst_r.at[half])

        for t in range(GRP):
          pltpu.async_copy(
              bufs[1], acc.at[dst_r.at[half, t]], sem_s[half], add=True)

    for half in (0, 1):
      for t in range(GRP):
        pltpu.make_async_copy(
            bufs[1], acc.at[dst_r.at[half, t]], sem_s[half]).wait()
    plsc.subcore_barrier()

    @pl.loop(0, rows_per_tile // LANE)
    def _publish_cnt(q):
      pltpu.sync_copy(acc.at[pl.ds(base + q * LANE, LANE)], bufs[2])
      pltpu.sync_copy(bufs[2], cnt_hbm.at[c, pl.ds(base + q * LANE, LANE)])

    @pl.loop(0, rows_per_tile // LANE)
    def _rezero_acc(q):
      pltpu.sync_copy(bufs[0], acc.at[pl.ds(base + q * LANE, LANE)])

    pltpu.sync_copy(src_hbm.at[wid, pl.ds(0, GRP)], src_r.at[0])
    pltpu.sync_copy(dst_hbm.at[wid, pl.ds(0, GRP)], dst_r.at[0])
    pltpu.sync_copy(src_hbm.at[wid, pl.ds(GRP, GRP)], src_r.at[1])
    pltpu.sync_copy(dst_hbm.at[wid, pl.ds(GRP, GRP)], dst_r.at[1])

    for t in range(NBUF - 1):
      pltpu.async_copy(x_hbm.at[src_r.at[0, t]], bufs[t], sem_g[t])
    plsc.subcore_barrier()

    @pl.loop(0, ngroups)
    def _group(g):
      b = g % 2
      nb = 1 - b
      for t in range(GRP):
        buf = bufs[t % NBUF]
        pltpu.make_async_copy(
            x_hbm.at[src_r.at[b, t]], buf, sem_g[t % NBUF]).wait()
        pltpu.async_copy(buf, acc.at[dst_r.at[b, t]], sem_s[t % NBUF],
                         add=True)
        if t > 0:
          pbuf = bufs[(t - 1) % NBUF]
          pltpu.make_async_copy(
              pbuf, acc.at[dst_r.at[b, t - 1]], sem_s[(t - 1) % NBUF]).wait()
          if t + NBUF - 1 < GRP:
            pltpu.async_copy(
                x_hbm.at[src_r.at[b, t + NBUF - 1]], pbuf,
                sem_g[(t - 1) % NBUF])
          else:
            @pl.when(g < ngroups - 1)
            def _start_next():
              pltpu.async_copy(
                  x_hbm.at[src_r.at[nb, t + NBUF - 1 - GRP]], pbuf,
                  sem_g[(t - 1) % NBUF])
        else:
          pltpu.async_copy(
              x_hbm.at[src_r.at[b, NBUF - 1]], bufs[NBUF - 1],
              sem_g[NBUF - 1])

      pltpu.make_async_copy(
          bufs[(GRP - 1) % NBUF], acc.at[dst_r.at[b, GRP - 1]],
          sem_s[(GRP - 1) % NBUF]).wait()

      @pl.when(g + 2 < ngroups)
      def _refill():
        off = (g + 2) * GRP
        pltpu.sync_copy(src_hbm.at[wid, pl.ds(off, GRP)], src_r.at[b])
        pltpu.sync_copy(dst_hbm.at[wid, pl.ds(off, GRP)], dst_r.at[b])

    plsc.subcore_barrier()

    @pl.loop(0, rows_per_tile // LANE)
    def _publish(q):
      pltpu.sync_copy(acc.at[pl.ds(base + q * LANE, LANE)], bufs[0])
      pltpu.sync_copy(bufs[0], out_hbm.at[c, pl.ds(base + q * LANE, LANE)])

  return pl.kernel(
      body,
      out_type=(
          jax.ShapeDtypeStruct((NC, n_pad, width), jnp.float32),
          jax.ShapeDtypeStruct((NC, n_pad, width), jnp.float32),
      ),
      mesh=mesh,
      scratch_types=(
          [
              pltpu.VMEM((2, GRP, LANE), jnp.int32),
              pltpu.VMEM((2, GRP, LANE), jnp.int32),
          ]
          + [pltpu.VMEM((LANE, width), jnp.float32) for _ in range(NBUF)]
          + [pltpu.SemaphoreType.DMA] * (2 * NBUF)
          + [pltpu.VMEM_SHARED((n_pad, width), jnp.float32)]
      ),
  )


def _dot_t(a, w):
  return lax.dot_general(a, w, (((1,), (1,)), ((), ())),
                         preferred_element_type=jnp.float32)


def _tc1_body(p_ref, c_ref, x_ref, wl_ref, wr_ref, b_ref, h_ref, inv_ref):
  p = p_ref[...]
  agg = p[0] + p[1]
  cnt = c_ref[0, :, 0:1] + c_ref[1, :, 0:1]
  inv = 1.0 / jnp.maximum(cnt, 1.0)
  inv_ref[...] = inv
  mean = agg * inv
  h = _dot_t(mean, wl_ref[...]) + b_ref[...] + _dot_t(x_ref[...], wr_ref[...])
  h_ref[...] = jnp.maximum(h, 0.0)


def _tc2_body(p_ref, inv_ref, h_ref, wl_ref, wr_ref, b_ref, o_ref):
  p = p_ref[...]
  agg = p[0] + p[1]
  mean = agg * inv_ref[...]
  z = _dot_t(mean, wl_ref[...]) + b_ref[...] + _dot_t(h_ref[...], wr_ref[...])
  z = jnp.maximum(z, 0.0)
  m = jnp.max(z, axis=1, keepdims=True)
  lse = jnp.log(jnp.sum(jnp.exp(z - m), axis=1, keepdims=True)) + m
  o_ref[...] = z - lse


def kernel(x, edge_index, W1_l, b1_l, W1_r, W2_l, b2_l, W2_r):
  n, d = x.shape
  e = edge_index.shape[1]
  nw = NC * NS

  nsteps = -(-e // (nw * LANE * GRP)) * GRP
  e_pad = nw * nsteps * LANE
  n_pad = -(-(n + 1) // (NS * LANE)) * (NS * LANE)

  src = edge_index[0].astype(jnp.int32)
  dst = edge_index[1].astype(jnp.int32)
  pad_idx = jnp.arange(e_pad - e, dtype=jnp.int32)
  src = jnp.concatenate([src, pad_idx % n])
  dst = jnp.concatenate([dst, n + pad_idx % (n_pad - n)])
  src3 = src.reshape(nw, nsteps, LANE)
  dst3 = dst.reshape(nw, nsteps, LANE)

  parts1, cnt_parts = _sc_cnt_agg(n_pad, d, nsteps)(x, src3, dst3)

  rb = 2048
  grid = (-(-n // rb),)
  b1_2d = b1_l.reshape(1, d)
  b2_2d = b2_l.reshape(1, d)

  h, invcol = pl.pallas_call(
      _tc1_body,
      grid=grid,
      in_specs=[
          pl.BlockSpec((NC, rb, d), lambda j: (0, j, 0)),
          pl.BlockSpec((NC, rb, d), lambda j: (0, j, 0)),
          pl.BlockSpec((rb, d), lambda j: (j, 0)),
          pl.BlockSpec((d, d), lambda j: (0, 0)),
          pl.BlockSpec((d, d), lambda j: (0, 0)),
          pl.BlockSpec((1, d), lambda j: (0, 0)),
      ],
      out_specs=[
          pl.BlockSpec((rb, d), lambda j: (j, 0)),
          pl.BlockSpec((rb, 1), lambda j: (j, 0)),
      ],
      out_shape=[
          jax.ShapeDtypeStruct((n, d), jnp.float32),
          jax.ShapeDtypeStruct((n_pad, 1), jnp.float32),
      ],
  )(parts1, cnt_parts, x, W1_l, W1_r, b1_2d)

  parts2 = _sc_agg(n_pad, d, nsteps)(h, src3, dst3)

  out = pl.pallas_call(
      _tc2_body,
      grid=grid,
      in_specs=[
          pl.BlockSpec((NC, rb, d), lambda j: (0, j, 0)),
          pl.BlockSpec((rb, 1), lambda j: (j, 0)),
          pl.BlockSpec((rb, d), lambda j: (j, 0)),
          pl.BlockSpec((d, d), lambda j: (0, 0)),
          pl.BlockSpec((d, d), lambda j: (0, 0)),
          pl.BlockSpec((1, d), lambda j: (0, 0)),
      ],
      out_specs=pl.BlockSpec((rb, d), lambda j: (j, 0)),
      out_shape=jax.ShapeDtypeStruct((n, d), jnp.float32),
  )(parts2, invcol, h, W2_l, W2_r, b2_2d)

  return out


# speedup vs baseline: 1.0062x; 1.0038x over previous
import jax
import jax.numpy as jnp
from jax import lax
from jax.experimental import pallas as pl
from jax.experimental.pallas import tpu as pltpu
from jax.experimental.pallas import tpu_sc as plsc

NC = 2
NS = 16
LANE = 64
GRP = 16
NBUF = 4


def _sc_agg(n_pad: int, width: int, nsteps: int):
  rows_per_tile = n_pad // NS
  ngroups = nsteps // GRP
  mesh = plsc.VectorSubcoreMesh(
      core_axis_name="c", subcore_axis_name="s", num_cores=NC, num_subcores=NS)

  def body(x_hbm, src_hbm, dst_hbm, out_hbm, src_r, dst_r, *rest):
    bufs = rest[:NBUF]
    sem_g = rest[NBUF:2 * NBUF]
    sem_s = rest[2 * NBUF:3 * NBUF]
    acc = rest[3 * NBUF]
    c = lax.axis_index("c")
    s = lax.axis_index("s")
    wid = c * NS + s
    base = s * rows_per_tile

    zv = jnp.zeros((16,), jnp.float32)

    @pl.loop(0, LANE)
    def _zero_buf(i):
      for j in range(width // 16):
        bufs[0][i, pl.ds(j * 16, 16)] = zv

    @pl.loop(0, rows_per_tile // LANE)
    def _zero_acc(q):
      pltpu.sync_copy(bufs[0], acc.at[pl.ds(base + q * LANE, LANE)])

    pltpu.sync_copy(src_hbm.at[wid, pl.ds(0, GRP)], src_r.at[0])
    pltpu.sync_copy(dst_hbm.at[wid, pl.ds(0, GRP)], dst_r.at[0])
    pltpu.sync_copy(src_hbm.at[wid, pl.ds(GRP, GRP)], src_r.at[1])
    pltpu.sync_copy(dst_hbm.at[wid, pl.ds(GRP, GRP)], dst_r.at[1])

    for t in range(NBUF - 1):
      pltpu.async_copy(x_hbm.at[src_r.at[0, t]], bufs[t], sem_g[t])
    plsc.subcore_barrier()

    @pl.loop(0, ngroups)
    def _group(g):
      b = g % 2
      nb = 1 - b
      for t in range(GRP):
        buf = bufs[t % NBUF]
        pltpu.make_async_copy(
            x_hbm.at[src_r.at[b, t]], buf, sem_g[t % NBUF]).wait()
        pltpu.async_copy(buf, acc.at[dst_r.at[b, t]], sem_s[t % NBUF],
                         add=True)
        if t > 0:
          pbuf = bufs[(t - 1) % NBUF]
          pltpu.make_async_copy(
              pbuf, acc.at[dst_r.at[b, t - 1]], sem_s[(t - 1) % NBUF]).wait()
          if t + NBUF - 1 < GRP:
            pltpu.async_copy(
                x_hbm.at[src_r.at[b, t + NBUF - 1]], pbuf,
                sem_g[(t - 1) % NBUF])
          else:
            @pl.when(g < ngroups - 1)
            def _start_next():
              pltpu.async_copy(
                  x_hbm.at[src_r.at[nb, t + NBUF - 1 - GRP]], pbuf,
                  sem_g[(t - 1) % NBUF])
        else:
          pltpu.async_copy(
              x_hbm.at[src_r.at[b, NBUF - 1]], bufs[NBUF - 1],
              sem_g[NBUF - 1])

      pltpu.make_async_copy(
          bufs[(GRP - 1) % NBUF], acc.at[dst_r.at[b, GRP - 1]],
          sem_s[(GRP - 1) % NBUF]).wait()

      @pl.when(g + 2 < ngroups)
      def _refill():
        off = (g + 2) * GRP
        pltpu.sync_copy(src_hbm.at[wid, pl.ds(off, GRP)], src_r.at[b])
        pltpu.sync_copy(dst_hbm.at[wid, pl.ds(off, GRP)], dst_r.at[b])

    plsc.subcore_barrier()

    @pl.loop(0, rows_per_tile // LANE)
    def _publish(q):
      pltpu.sync_copy(acc.at[pl.ds(base + q * LANE, LANE)],
                      out_hbm.at[c, pl.ds(base + q * LANE, LANE)])

  return pl.kernel(
      body,
      out_type=jax.ShapeDtypeStruct((NC, n_pad, width), jnp.float32),
      mesh=mesh,
      scratch_types=(
          [
              pltpu.VMEM((2, GRP, LANE), jnp.int32),
              pltpu.VMEM((2, GRP, LANE), jnp.int32),
          ]
          + [pltpu.VMEM((LANE, width), jnp.float32) for _ in range(NBUF)]
          + [pltpu.SemaphoreType.DMA] * (2 * NBUF)
          + [pltpu.VMEM_SHARED((n_pad, width), jnp.float32)]
      ),
  )


def _sc_cnt_agg(n_pad: int, width: int, nsteps: int):
  rows_per_tile = n_pad // NS
  ngroups = nsteps // GRP
  mesh = plsc.VectorSubcoreMesh(
      core_axis_name="c", subcore_axis_name="s", num_cores=NC, num_subcores=NS)

  def body(x_hbm, src_hbm, dst_hbm, out_hbm, cnt_hbm, src_r, dst_r, *rest):
    bufs = rest[:NBUF]
    sem_g = rest[NBUF:2 * NBUF]
    sem_s = rest[2 * NBUF:3 * NBUF]
    acc = rest[3 * NBUF]
    c = lax.axis_index("c")
    s = lax.axis_index("s")
    wid = c * NS + s
    base = s * rows_per_tile

    zv = jnp.zeros((16,), jnp.float32)
    ov = jnp.ones((16,), jnp.float32)

    @pl.loop(0, LANE)
    def _fill(i):
      for j in range(width // 16):
        bufs[0][i, pl.ds(j * 16, 16)] = zv
        bufs[1][i, pl.ds(j * 16, 16)] = ov

    @pl.loop(0, rows_per_tile // LANE)
    def _zero_acc(q):
      pltpu.sync_copy(bufs[0], acc.at[pl.ds(base + q * LANE, LANE)])

    pltpu.sync_copy(dst_hbm.at[wid, pl.ds(0, GRP)], dst_r.at[0])
    pltpu.sync_copy(dst_hbm.at[wid, pl.ds(GRP, GRP)], dst_r.at[1])
    plsc.subcore_barrier()

    @pl.loop(0, ngroups // 2)
    def _cnt_pair(k):
      for half in (0, 1):
        @pl.when(k >= 1)
        def _recycle(half=half):
          for t in range(GRP):
            pltpu.make_async_copy(
                bufs[1], acc.at[dst_r.at[half, t]], sem_s[half]).wait()
          pltpu.sync_copy(
              dst_hbm.at[wid, pl.ds((2 * k + half) * GRP, GRP)],
              dst_r.at[half])

        for t in range(GRP):
          pltpu.async_copy(
              bufs[1], acc.at[dst_r.at[half, t]], sem_s[half], add=True)

    for half in (0, 1):
      for t in range(GRP):
        pltpu.make_async_copy(
            bufs[1], acc.at[dst_r.at[half, t]], sem_s[half]).wait()
    plsc.subcore_barrier()

    @pl.loop(0, rows_per_tile // LANE)
    def _publish_cnt(q):
      pltpu.sync_copy(acc.at[pl.ds(base + q * LANE, LANE)],
                      cnt_hbm.at[c, pl.ds(base + q * LANE, LANE)])

    @pl.loop(0, rows_per_tile // LANE)
    def _rezero_acc(q):
      pltpu.sync_copy(bufs[0], acc.at[pl.ds(base + q * LANE, LANE)])

    pltpu.sync_copy(src_hbm.at[wid, pl.ds(0, GRP)], src_r.at[0])
    pltpu.sync_copy(dst_hbm.at[wid, pl.ds(0, GRP)], dst_r.at[0])
    pltpu.sync_copy(src_hbm.at[wid, pl.ds(GRP, GRP)], src_r.at[1])
    pltpu.sync_copy(dst_hbm.at[wid, pl.ds(GRP, GRP)], dst_r.at[1])

    for t in range(NBUF - 1):
      pltpu.async_copy(x_hbm.at[src_r.at[0, t]], bufs[t], sem_g[t])
    plsc.subcore_barrier()

    @pl.loop(0, ngroups)
    def _group(g):
      b = g % 2
      nb = 1 - b
      for t in range(GRP):
        buf = bufs[t % NBUF]
        pltpu.make_async_copy(
            x_hbm.at[src_r.at[b, t]], buf, sem_g[t % NBUF]).wait()
        pltpu.async_copy(buf, acc.at[dst_r.at[b, t]], sem_s[t % NBUF],
                         add=True)
        if t > 0:
          pbuf = bufs[(t - 1) % NBUF]
          pltpu.make_async_copy(
              pbuf, acc.at[dst_r.at[b, t - 1]], sem_s[(t - 1) % NBUF]).wait()
          if t + NBUF - 1 < GRP:
            pltpu.async_copy(
                x_hbm.at[src_r.at[b, t + NBUF - 1]], pbuf,
                sem_g[(t - 1) % NBUF])
          else:
            @pl.when(g < ngroups - 1)
            def _start_next():
              pltpu.async_copy(
                  x_hbm.at[src_r.at[nb, t + NBUF - 1 - GRP]], pbuf,
                  sem_g[(t - 1) % NBUF])
        else:
          pltpu.async_copy(
              x_hbm.at[src_r.at[b, NBUF - 1]], bufs[NBUF - 1],
              sem_g[NBUF - 1])

      pltpu.make_async_copy(
          bufs[(GRP - 1) % NBUF], acc.at[dst_r.at[b, GRP - 1]],
          sem_s[(GRP - 1) % NBUF]).wait()

      @pl.when(g + 2 < ngroups)
      def _refill():
        off = (g + 2) * GRP
        pltpu.sync_copy(src_hbm.at[wid, pl.ds(off, GRP)], src_r.at[b])
        pltpu.sync_copy(dst_hbm.at[wid, pl.ds(off, GRP)], dst_r.at[b])

    plsc.subcore_barrier()

    @pl.loop(0, rows_per_tile // LANE)
    def _publish(q):
      pltpu.sync_copy(acc.at[pl.ds(base + q * LANE, LANE)],
                      out_hbm.at[c, pl.ds(base + q * LANE, LANE)])

  return pl.kernel(
      body,
      out_type=(
          jax.ShapeDtypeStruct((NC, n_pad, width), jnp.float32),
          jax.ShapeDtypeStruct((NC, n_pad, width), jnp.float32),
      ),
      mesh=mesh,
      scratch_types=(
          [
              pltpu.VMEM((2, GRP, LANE), jnp.int32),
              pltpu.VMEM((2, GRP, LANE), jnp.int32),
          ]
          + [pltpu.VMEM((LANE, width), jnp.float32) for _ in range(NBUF)]
          + [pltpu.SemaphoreType.DMA] * (2 * NBUF)
          + [pltpu.VMEM_SHARED((n_pad, width), jnp.float32)]
      ),
  )


def _dot_t(a, w):
  return lax.dot_general(a, w, (((1,), (1,)), ((), ())),
                         preferred_element_type=jnp.float32)


def _tc1_body(p_ref, c_ref, x_ref, wl_ref, wr_ref, b_ref, h_ref, inv_ref):
  p = p_ref[...]
  agg = p[0] + p[1]
  cnt = c_ref[0, :, 0:1] + c_ref[1, :, 0:1]
  inv = 1.0 / jnp.maximum(cnt, 1.0)
  inv_ref[...] = inv
  mean = agg * inv
  h = _dot_t(mean, wl_ref[...]) + b_ref[...] + _dot_t(x_ref[...], wr_ref[...])
  h_ref[...] = jnp.maximum(h, 0.0)


def _tc2_body(p_ref, inv_ref, h_ref, wl_ref, wr_ref, b_ref, o_ref):
  p = p_ref[...]
  agg = p[0] + p[1]
  mean = agg * inv_ref[...]
  z = _dot_t(mean, wl_ref[...]) + b_ref[...] + _dot_t(h_ref[...], wr_ref[...])
  z = jnp.maximum(z, 0.0)
  m = jnp.max(z, axis=1, keepdims=True)
  lse = jnp.log(jnp.sum(jnp.exp(z - m), axis=1, keepdims=True)) + m
  o_ref[...] = z - lse


def kernel(x, edge_index, W1_l, b1_l, W1_r, W2_l, b2_l, W2_r):
  n, d = x.shape
  e = edge_index.shape[1]
  nw = NC * NS

  nsteps = -(-e // (nw * LANE * GRP)) * GRP
  e_pad = nw * nsteps * LANE
  n_pad = -(-(n + 1) // (NS * LANE)) * (NS * LANE)

  src = edge_index[0].astype(jnp.int32)
  dst = edge_index[1].astype(jnp.int32)
  pad_idx = jnp.arange(e_pad - e, dtype=jnp.int32)
  src = jnp.concatenate([src, pad_idx % n])
  dst = jnp.concatenate([dst, n + pad_idx % (n_pad - n)])
  src3 = src.reshape(nw, nsteps, LANE)
  dst3 = dst.reshape(nw, nsteps, LANE)

  parts1, cnt_parts = _sc_cnt_agg(n_pad, d, nsteps)(x, src3, dst3)

  rb = 2048
  grid = (-(-n // rb),)
  b1_2d = b1_l.reshape(1, d)
  b2_2d = b2_l.reshape(1, d)

  h, invcol = pl.pallas_call(
      _tc1_body,
      grid=grid,
      in_specs=[
          pl.BlockSpec((NC, rb, d), lambda j: (0, j, 0)),
          pl.BlockSpec((NC, rb, d), lambda j: (0, j, 0)),
          pl.BlockSpec((rb, d), lambda j: (j, 0)),
          pl.BlockSpec((d, d), lambda j: (0, 0)),
          pl.BlockSpec((d, d), lambda j: (0, 0)),
          pl.BlockSpec((1, d), lambda j: (0, 0)),
      ],
      out_specs=[
          pl.BlockSpec((rb, d), lambda j: (j, 0)),
          pl.BlockSpec((rb, 1), lambda j: (j, 0)),
      ],
      out_shape=[
          jax.ShapeDtypeStruct((n, d), jnp.float32),
          jax.ShapeDtypeStruct((n_pad, 1), jnp.float32),
      ],
  )(parts1, cnt_parts, x, W1_l, W1_r, b1_2d)

  parts2 = _sc_agg(n_pad, d, nsteps)(h, src3, dst3)

  out = pl.pallas_call(
      _tc2_body,
      grid=grid,
      in_specs=[
          pl.BlockSpec((NC, rb, d), lambda j: (0, j, 0)),
          pl.BlockSpec((rb, 1), lambda j: (j, 0)),
          pl.BlockSpec((rb, d), lambda j: (j, 0)),
          pl.BlockSpec((d, d), lambda j: (0, 0)),
          pl.BlockSpec((d, d), lambda j: (0, 0)),
          pl.BlockSpec((1, d), lambda j: (0, 0)),
      ],
      out_specs=pl.BlockSpec((rb, d), lambda j: (j, 0)),
      out_shape=jax.ShapeDtypeStruct((n, d), jnp.float32),
  )(parts2, invcol, h, W2_l, W2_r, b2_2d)

  return out
